# Initial kernel scaffold; baseline (speedup 1.0000x reference)
import jax
import jax.numpy as jnp
from jax.experimental import pallas as pl


def kernel(x, edge_index, pre_W1, pre_b1, pre_W2, pre_b2, conv_W, conv_b, ffn_W1, ffn_b1, ffn_W2, ffn_b2, ln_g, ln_b, post_W1, post_b1, post_W2, post_b2):
    raise NotImplementedError("write your pallas kernel here")


# speedup vs baseline: 16.9288x; 16.9288x over previous
import functools

import jax
import jax.numpy as jnp
from jax import lax
from jax.experimental import pallas as pl
from jax.experimental.pallas import tpu as pltpu
from jax.experimental.pallas import tpu_sc as plsc

N = 10000
E = 320000
D = 128
HOPS_ = 5

NC, NS = 2, 16
NW = NC * NS
DH = D // NC
EPS = E // NS
C = 100
NCH = EPS // C
NP = 10240
NPS = NP // NS

ROWS = 1000


def _gelu(v):
    return 0.5 * v * (1.0 + lax.erf(v * 0.7071067811865476))


def _mm(a, b):
    return jax.lax.dot_general(a, b, (((1,), (0,)), ((), ())),
                               preferred_element_type=jnp.float32)


_MESH = plsc.VectorSubcoreMesh(core_axis_name="c", subcore_axis_name="s")


DCH = E // NW // C


def _deg_body(dst_hbm, ones_hbm, z8_hbm, out_hbm, dst_v, ones_v, acc_sh):
    c = lax.axis_index("c")
    s = lax.axis_index("s")
    wid = s * NC + c
    pltpu.sync_copy(dst_hbm.at[wid], dst_v)
    pltpu.sync_copy(ones_hbm, ones_v)
    pltpu.sync_copy(z8_hbm, acc_sh.at[pl.ds(s * NPS, NPS)])
    plsc.subcore_barrier()

    def body(j, carry):
        pltpu.sync_copy(ones_v, acc_sh.at[dst_v.at[j]], add=True)
        return carry

    lax.fori_loop(0, DCH, body, 0)
    plsc.subcore_barrier()
    pltpu.sync_copy(acc_sh.at[pl.ds(s * NPS, NPS)],
                    out_hbm.at[c, pl.ds(s * NPS, NPS)])


_deg_call = functools.partial(
    pl.kernel,
    _deg_body,
    out_type=jax.ShapeDtypeStruct((NC, NP, 8), jnp.float32),
    mesh=_MESH,
    compiler_params=pltpu.CompilerParams(use_tc_tiling_on_sc=False),
    scratch_types=[
        pltpu.VMEM((DCH, C), jnp.int32),
        pltpu.VMEM((C, 8), jnp.float32),
        pltpu.VMEM_SHARED((NP, 8), jnp.float32),
    ],
)()


def _hop_body(xws_hbm, src_hbm, dst_hbm, z_hbm, out_hbm,
              src_v, dst_v, rows_v, acc_sh, gsem, ssem):
    c = lax.axis_index("c")
    s = lax.axis_index("s")
    pltpu.sync_copy(src_hbm.at[s], src_v)
    pltpu.sync_copy(dst_hbm.at[s], dst_v)
    pltpu.sync_copy(z_hbm, acc_sh.at[pl.ds(s * NPS, NPS)])
    plsc.subcore_barrier()
    xc = xws_hbm.at[c]

    def start_gather(j, b):
        pltpu.async_copy(xc.at[src_v.at[j]], rows_v.at[b], gsem.at[b])

    def wait_gather(b):
        pltpu.make_async_copy(xc.at[src_v.at[0]], rows_v.at[b],
                              gsem.at[b]).wait()

    def start_scatter(j, b):
        pltpu.async_copy(rows_v.at[b], acc_sh.at[dst_v.at[j]], ssem.at[b],
                         add=True)

    def wait_scatter(b):
        pltpu.make_async_copy(rows_v.at[b], acc_sh.at[dst_v.at[0]],
                              ssem.at[b]).wait()

    start_gather(0, 0)

    def body(jj, carry):
        for b in range(2):
            j = 2 * jj + b

            @pl.when(j + 1 < NCH)
            def _():
                @pl.when(j >= 1)
                def _():
                    wait_scatter(1 - b)
                start_gather(j + 1, 1 - b)

            wait_gather(b)
            start_scatter(j, b)
        return carry

    lax.fori_loop(0, NCH // 2, body, 0)
    wait_scatter(0)
    wait_scatter(1)
    plsc.subcore_barrier()
    pltpu.sync_copy(acc_sh.at[pl.ds(s * NPS, NPS)],
                    out_hbm.at[c, pl.ds(s * NPS, NPS)])


_hop_call = functools.partial(
    pl.kernel,
    _hop_body,
    out_type=jax.ShapeDtypeStruct((NC, NP, DH), jnp.float32),
    mesh=_MESH,
    compiler_params=pltpu.CompilerParams(use_tc_tiling_on_sc=False),
    scratch_types=[
        pltpu.VMEM((NCH, C), jnp.int32),
        pltpu.VMEM((NCH, C), jnp.int32),
        pltpu.VMEM((2, C, DH), jnp.float32),
        pltpu.VMEM_SHARED((NP, DH), jnp.float32),
        pltpu.SemaphoreType.DMA((2,)),
        pltpu.SemaphoreType.DMA((2,)),
    ],
)()


def _full(shape):
    return pl.BlockSpec(shape, lambda i: (0,) * len(shape))


_W = _full((D, D))
_B = _full((1, D))
_ROWBLK = pl.BlockSpec((ROWS, D), lambda i: (i, 0))
_ACCBLK = pl.BlockSpec((NC, ROWS, DH), lambda i: (0, i, 0))
_DEGBLK = pl.BlockSpec((NC, ROWS, 8), lambda i: (0, i, 0))
_DINVBLK = pl.BlockSpec((ROWS, 8), lambda i: (i, 0))


def _stage0_body(x_ref, w1, b1, w2, b2, cw, h_ref, xw_ref):
    h = _mm(_gelu(_mm(x_ref[...], w1[...]) + b1[...]), w2[...]) + b2[...]
    h_ref[...] = h
    xw_ref[...] = _mm(h, cw[...])


_stage0 = pl.pallas_call(
    _stage0_body,
    grid=(N // ROWS,),
    in_specs=[_ROWBLK, _W, _B, _W, _B, _W],
    out_specs=[_ROWBLK, _ROWBLK],
    out_shape=[jax.ShapeDtypeStruct((N, D), jnp.float32)] * 2,
)


def _write_xws(xws_ref, v):
    xws_ref[0] = v[:, :DH]
    xws_ref[1] = v[:, DH:]


def _scale_body(degp, xw, dinv_ref, xws_ref):
    deg = degp[0] + degp[1] + 1.0
    dinv = lax.rsqrt(deg)
    dinv_ref[...] = dinv
    _write_xws(xws_ref, dinv[:, 0:1] * xw[...])


_scale = pl.pallas_call(
    _scale_body,
    grid=(N // ROWS,),
    in_specs=[_DEGBLK, _ROWBLK],
    out_specs=[_DINVBLK, _ACCBLK],
    out_shape=[jax.ShapeDtypeStruct((N, 8), jnp.float32),
               jax.ShapeDtypeStruct((NC, N, DH), jnp.float32)],
)


def _combine_ln_ffn(acc, xw, h_in, dinv, cb, g, b, fw1, fb1, fw2, fb2):
    di = dinv[:, 0:1]
    aggc = jnp.concatenate([acc[0], acc[1]], axis=-1)
    agg = aggc * di + (di * di) * xw + cb
    h1 = agg + h_in
    mu = jnp.mean(h1, axis=-1, keepdims=True)
    cen = h1 - mu
    var = jnp.mean(cen * cen, axis=-1, keepdims=True)
    hn = cen * lax.rsqrt(var + 1e-5) * g + b
    return _mm(_gelu(_mm(hn, fw1) + fb1), fw2) + fb2 + hn


def _hop_post_body(acc, xw, h, dinv, cb, g, b, fw1, fb1, fw2, fb2, cwn,
                   h_out, xw_out, xws_out):
    h2 = _combine_ln_ffn(acc[...], xw[...], h[...], dinv[...], cb[...],
                         g[...], b[...], fw1[...], fb1[...], fw2[...],
                         fb2[...])
    h_out[...] = h2
    xwn = _mm(h2, cwn[...])
    xw_out[...] = xwn
    _write_xws(xws_out, dinv[:, 0:1] * xwn)


_hop_post = pl.pallas_call(
    _hop_post_body,
    grid=(N // ROWS,),
    in_specs=[_ACCBLK, _ROWBLK, _ROWBLK, _DINVBLK,
              _B, _B, _B, _W, _B, _W, _B, _W],
    out_specs=[_ROWBLK, _ROWBLK, _ACCBLK],
    out_shape=[jax.ShapeDtypeStruct((N, D), jnp.float32),
               jax.ShapeDtypeStruct((N, D), jnp.float32),
               jax.ShapeDtypeStruct((NC, N, DH), jnp.float32)],
)


def _final_body(acc, xw, h, dinv, cb, g, b, fw1, fb1, fw2, fb2,
                pw1, pb1, pw2, pb2, out_ref):
    h2 = _combine_ln_ffn(acc[...], xw[...], h[...], dinv[...], cb[...],
                         g[...], b[...], fw1[...], fb1[...], fw2[...],
                         fb2[...])
    out_ref[...] = _mm(_gelu(_mm(h2, pw1[...]) + pb1[...]), pw2[...]) + pb2[...]


_final = pl.pallas_call(
    _final_body,
    grid=(N // ROWS,),
    in_specs=[_ACCBLK, _ROWBLK, _ROWBLK, _DINVBLK,
              _B, _B, _B, _W, _B, _W, _B, _W, _B, _W, _B],
    out_specs=_ROWBLK,
    out_shape=jax.ShapeDtypeStruct((N, D), jnp.float32),
)


def kernel(x, edge_index, pre_W1, pre_b1, pre_W2, pre_b2, conv_W, conv_b,
           ffn_W1, ffn_b1, ffn_W2, ffn_b2, ln_g, ln_b, post_W1, post_b1,
           post_W2, post_b2):
    f32 = jnp.float32
    src3 = edge_index[0].reshape(NS, NCH, C)
    dst3 = edge_index[1].reshape(NS, NCH, C)
    dstd = edge_index[1].reshape(NW, DCH, C)
    zrow = jnp.zeros((NPS, DH), f32)
    z8 = jnp.zeros((NPS, 8), f32)
    ones8 = jnp.ones((C, 8), f32)

    def r1(v):
        return v.reshape(1, D).astype(f32)

    degp = _deg_call(dstd, ones8, z8)
    h, xw = _stage0(x.astype(f32), pre_W1, r1(pre_b1), pre_W2, r1(pre_b2),
                    conv_W[0])
    dinv, xws = _scale(degp, xw)

    for i in range(HOPS_):
        acc = _hop_call(xws, src3, dst3, zrow)
        if i + 1 < HOPS_:
            h, xw, xws = _hop_post(
                acc, xw, h, dinv, r1(conv_b[i]), r1(ln_g[i]), r1(ln_b[i]),
                ffn_W1[i], r1(ffn_b1[i]), ffn_W2[i], r1(ffn_b2[i]),
                conv_W[i + 1])
        else:
            out = _final(
                acc, xw, h, dinv, r1(conv_b[i]), r1(ln_g[i]), r1(ln_b[i]),
                ffn_W1[i], r1(ffn_b1[i]), ffn_W2[i], r1(ffn_b2[i]),
                post_W1, r1(post_b1), post_W2, r1(post_b2))
    return out


# speedup vs baseline: 20.3890x; 1.2044x over previous
import functools

import jax
import jax.numpy as jnp
from jax import lax
from jax.experimental import pallas as pl
from jax.experimental.pallas import tpu as pltpu
from jax.experimental.pallas import tpu_sc as plsc

N = 10000
E = 320000
D = 128
HOPS_ = 5

NC, NS = 2, 16
NW = NC * NS
DH = D // NC
EPS = E // NS
C = 100
NCH = EPS // C
NP = 10240
NPS = NP // NS

ROWS = 1000


def _gelu(v):
    return 0.5 * v * (1.0 + lax.erf(v * 0.7071067811865476))


def _mm(a, b):
    return jax.lax.dot_general(a, b, (((1,), (0,)), ((), ())),
                               preferred_element_type=jnp.float32)


_MESH = plsc.VectorSubcoreMesh(core_axis_name="c", subcore_axis_name="s")


DCH = E // NW // C


def _deg_body(dst_hbm, ones_hbm, z8_hbm, out_hbm, dst_v, ones_v, acc_sh):
    c = lax.axis_index("c")
    s = lax.axis_index("s")
    wid = s * NC + c
    pltpu.sync_copy(dst_hbm.at[wid], dst_v)
    pltpu.sync_copy(ones_hbm, ones_v)
    pltpu.sync_copy(z8_hbm, acc_sh.at[pl.ds(s * NPS, NPS)])
    plsc.subcore_barrier()

    def body(j, carry):
        pltpu.sync_copy(ones_v, acc_sh.at[dst_v.at[j]], add=True)
        return carry

    lax.fori_loop(0, DCH, body, 0)
    plsc.subcore_barrier()
    pltpu.sync_copy(acc_sh.at[pl.ds(s * NPS, NPS)],
                    out_hbm.at[c, pl.ds(s * NPS, NPS)])


_deg_call = functools.partial(
    pl.kernel,
    _deg_body,
    out_type=jax.ShapeDtypeStruct((NC, NP, 8), jnp.float32),
    mesh=_MESH,
    compiler_params=pltpu.CompilerParams(use_tc_tiling_on_sc=False),
    scratch_types=[
        pltpu.VMEM((DCH, C), jnp.int32),
        pltpu.VMEM((C, 8), jnp.float32),
        pltpu.VMEM_SHARED((NP, 8), jnp.float32),
    ],
)()


def _hop_body(xws_hbm, src_hbm, dst_hbm, z_hbm, out_hbm,
              src_v, dst_v, rows_v, acc_sh, gsem, ssem):
    c = lax.axis_index("c")
    s = lax.axis_index("s")
    pltpu.sync_copy(src_hbm.at[s], src_v)
    pltpu.sync_copy(dst_hbm.at[s], dst_v)
    pltpu.sync_copy(z_hbm, acc_sh.at[pl.ds(s * NPS, NPS)])
    plsc.subcore_barrier()
    xc = xws_hbm.at[c]

    def start_gather(j, b):
        pltpu.async_copy(xc.at[src_v.at[j]], rows_v.at[b], gsem.at[b])

    def wait_gather(b):
        pltpu.make_async_copy(xc.at[src_v.at[0]], rows_v.at[b],
                              gsem.at[b]).wait()

    def start_scatter(j, b):
        pltpu.async_copy(rows_v.at[b], acc_sh.at[dst_v.at[j]], ssem.at[b],
                         add=True)

    def wait_scatter(b):
        pltpu.make_async_copy(rows_v.at[b], acc_sh.at[dst_v.at[0]],
                              ssem.at[b]).wait()

    start_gather(0, 0)
    start_gather(1, 1)

    def body(jj, carry):
        for b in range(4):
            j = 4 * jj + b
            bn = (b + 2) % 4

            @pl.when(j + 2 < NCH)
            def _():
                @pl.when(j >= 2)
                def _():
                    wait_scatter(bn)
                start_gather(j + 2, bn)

            wait_gather(b)
            start_scatter(j, b)
        return carry

    lax.fori_loop(0, NCH // 4, body, 0)
    for b in range(4):
        wait_scatter(b)
    plsc.subcore_barrier()
    pltpu.sync_copy(acc_sh.at[pl.ds(s * NPS, NPS)],
                    out_hbm.at[c, pl.ds(s * NPS, NPS)])


_hop_call = functools.partial(
    pl.kernel,
    _hop_body,
    out_type=jax.ShapeDtypeStruct((NC, NP, DH), jnp.float32),
    mesh=_MESH,
    compiler_params=pltpu.CompilerParams(use_tc_tiling_on_sc=False),
    scratch_types=[
        pltpu.VMEM((NCH, C), jnp.int32),
        pltpu.VMEM((NCH, C), jnp.int32),
        pltpu.VMEM((4, C, DH), jnp.float32),
        pltpu.VMEM_SHARED((NP, DH), jnp.float32),
        pltpu.SemaphoreType.DMA((4,)),
        pltpu.SemaphoreType.DMA((4,)),
    ],
)()


def _full(shape):
    return pl.BlockSpec(shape, lambda i: (0,) * len(shape))


_W = _full((D, D))
_B = _full((1, D))
_ROWBLK = pl.BlockSpec((ROWS, D), lambda i: (i, 0))
_ACCBLK = pl.BlockSpec((NC, ROWS, DH), lambda i: (0, i, 0))
_DEGBLK = pl.BlockSpec((NC, ROWS, 8), lambda i: (0, i, 0))
_DINVBLK = pl.BlockSpec((ROWS, 8), lambda i: (i, 0))


def _stage0_body(x_ref, w1, b1, w2, b2, cw, h_ref, xw_ref):
    h = _mm(_gelu(_mm(x_ref[...], w1[...]) + b1[...]), w2[...]) + b2[...]
    h_ref[...] = h
    xw_ref[...] = _mm(h, cw[...])


_stage0 = pl.pallas_call(
    _stage0_body,
    grid=(N // ROWS,),
    in_specs=[_ROWBLK, _W, _B, _W, _B, _W],
    out_specs=[_ROWBLK, _ROWBLK],
    out_shape=[jax.ShapeDtypeStruct((N, D), jnp.float32)] * 2,
)


def _write_xws(xws_ref, v):
    xws_ref[0] = v[:, :DH]
    xws_ref[1] = v[:, DH:]


def _scale_body(degp, xw, dinv_ref, xws_ref):
    deg = degp[0] + degp[1] + 1.0
    dinv = lax.rsqrt(deg)
    dinv_ref[...] = dinv
    _write_xws(xws_ref, dinv[:, 0:1] * xw[...])


_scale = pl.pallas_call(
    _scale_body,
    grid=(N // ROWS,),
    in_specs=[_DEGBLK, _ROWBLK],
    out_specs=[_DINVBLK, _ACCBLK],
    out_shape=[jax.ShapeDtypeStruct((N, 8), jnp.float32),
               jax.ShapeDtypeStruct((NC, N, DH), jnp.float32)],
)


def _combine_ln_ffn(acc, xw, h_in, dinv, cb, g, b, fw1, fb1, fw2, fb2):
    di = dinv[:, 0:1]
    aggc = jnp.concatenate([acc[0], acc[1]], axis=-1)
    agg = aggc * di + (di * di) * xw + cb
    h1 = agg + h_in
    mu = jnp.mean(h1, axis=-1, keepdims=True)
    cen = h1 - mu
    var = jnp.mean(cen * cen, axis=-1, keepdims=True)
    hn = cen * lax.rsqrt(var + 1e-5) * g + b
    return _mm(_gelu(_mm(hn, fw1) + fb1), fw2) + fb2 + hn


def _hop_post_body(acc, xw, h, dinv, cb, g, b, fw1, fb1, fw2, fb2, cwn,
                   h_out, xw_out, xws_out):
    h2 = _combine_ln_ffn(acc[...], xw[...], h[...], dinv[...], cb[...],
                         g[...], b[...], fw1[...], fb1[...], fw2[...],
                         fb2[...])
    h_out[...] = h2
    xwn = _mm(h2, cwn[...])
    xw_out[...] = xwn
    _write_xws(xws_out, dinv[:, 0:1] * xwn)


_hop_post = pl.pallas_call(
    _hop_post_body,
    grid=(N // ROWS,),
    in_specs=[_ACCBLK, _ROWBLK, _ROWBLK, _DINVBLK,
              _B, _B, _B, _W, _B, _W, _B, _W],
    out_specs=[_ROWBLK, _ROWBLK, _ACCBLK],
    out_shape=[jax.ShapeDtypeStruct((N, D), jnp.float32),
               jax.ShapeDtypeStruct((N, D), jnp.float32),
               jax.ShapeDtypeStruct((NC, N, DH), jnp.float32)],
)


def _final_body(acc, xw, h, dinv, cb, g, b, fw1, fb1, fw2, fb2,
                pw1, pb1, pw2, pb2, out_ref):
    h2 = _combine_ln_ffn(acc[...], xw[...], h[...], dinv[...], cb[...],
                         g[...], b[...], fw1[...], fb1[...], fw2[...],
                         fb2[...])
    out_ref[...] = _mm(_gelu(_mm(h2, pw1[...]) + pb1[...]), pw2[...]) + pb2[...]


_final = pl.pallas_call(
    _final_body,
    grid=(N // ROWS,),
    in_specs=[_ACCBLK, _ROWBLK, _ROWBLK, _DINVBLK,
              _B, _B, _B, _W, _B, _W, _B, _W, _B, _W, _B],
    out_specs=_ROWBLK,
    out_shape=jax.ShapeDtypeStruct((N, D), jnp.float32),
)


def kernel(x, edge_index, pre_W1, pre_b1, pre_W2, pre_b2, conv_W, conv_b,
           ffn_W1, ffn_b1, ffn_W2, ffn_b2, ln_g, ln_b, post_W1, post_b1,
           post_W2, post_b2):
    f32 = jnp.float32
    src3 = edge_index[0].reshape(NS, NCH, C)
    dst3 = edge_index[1].reshape(NS, NCH, C)
    dstd = edge_index[1].reshape(NW, DCH, C)
    zrow = jnp.zeros((NPS, DH), f32)
    z8 = jnp.zeros((NPS, 8), f32)
    ones8 = jnp.ones((C, 8), f32)

    def r1(v):
        return v.reshape(1, D).astype(f32)

    degp = _deg_call(dstd, ones8, z8)
    h, xw = _stage0(x.astype(f32), pre_W1, r1(pre_b1), pre_W2, r1(pre_b2),
                    conv_W[0])
    dinv, xws = _scale(degp, xw)

    for i in range(HOPS_):
        acc = _hop_call(xws, src3, dst3, zrow)
        if i + 1 < HOPS_:
            h, xw, xws = _hop_post(
                acc, xw, h, dinv, r1(conv_b[i]), r1(ln_g[i]), r1(ln_b[i]),
                ffn_W1[i], r1(ffn_b1[i]), ffn_W2[i], r1(ffn_b2[i]),
                conv_W[i + 1])
        else:
            out = _final(
                acc, xw, h, dinv, r1(conv_b[i]), r1(ln_g[i]), r1(ln_b[i]),
                ffn_W1[i], r1(ffn_b1[i]), ffn_W2[i], r1(ffn_b2[i]),
                post_W1, r1(post_b1), post_W2, r1(post_b2))
    return out


# speedup vs baseline: 20.4496x; 1.0030x over previous
import functools

import jax
import jax.numpy as jnp
from jax import lax
from jax.experimental import pallas as pl
from jax.experimental.pallas import tpu as pltpu
from jax.experimental.pallas import tpu_sc as plsc

N = 10000
E = 320000
D = 128
HOPS_ = 5

NC, NS = 2, 16
NW = NC * NS
DH = D // NC
EPS = E // NS
C = 125
NCH = EPS // C
NP = 10240
NPS = NP // NS

ROWS = 1000


def _gelu(v):
    return 0.5 * v * (1.0 + lax.erf(v * 0.7071067811865476))


def _mm(a, b):
    return jax.lax.dot_general(a, b, (((1,), (0,)), ((), ())),
                               preferred_element_type=jnp.float32)


_MESH = plsc.VectorSubcoreMesh(core_axis_name="c", subcore_axis_name="s")


DCH = E // NW // C


def _deg_body(dst_hbm, ones_hbm, z8_hbm, out_hbm, dst_v, ones_v, acc_sh):
    c = lax.axis_index("c")
    s = lax.axis_index("s")
    wid = s * NC + c
    pltpu.sync_copy(dst_hbm.at[wid], dst_v)
    pltpu.sync_copy(ones_hbm, ones_v)
    pltpu.sync_copy(z8_hbm, acc_sh.at[pl.ds(s * NPS, NPS)])
    plsc.subcore_barrier()

    def body(j, carry):
        pltpu.sync_copy(ones_v, acc_sh.at[dst_v.at[j]], add=True)
        return carry

    lax.fori_loop(0, DCH, body, 0)
    plsc.subcore_barrier()
    pltpu.sync_copy(acc_sh.at[pl.ds(s * NPS, NPS)],
                    out_hbm.at[c, pl.ds(s * NPS, NPS)])


_deg_call = functools.partial(
    pl.kernel,
    _deg_body,
    out_type=jax.ShapeDtypeStruct((NC, NP, 8), jnp.float32),
    mesh=_MESH,
    compiler_params=pltpu.CompilerParams(use_tc_tiling_on_sc=False),
    scratch_types=[
        pltpu.VMEM((DCH, C), jnp.int32),
        pltpu.VMEM((C, 8), jnp.float32),
        pltpu.VMEM_SHARED((NP, 8), jnp.float32),
    ],
)()


def _hop_body(xws_hbm, src_hbm, dst_hbm, z_hbm, out_hbm,
              src_v, dst_v, rows_v, acc_sh, gsem, ssem):
    c = lax.axis_index("c")
    s = lax.axis_index("s")
    pltpu.sync_copy(src_hbm.at[s], src_v)
    pltpu.sync_copy(dst_hbm.at[s], dst_v)
    pltpu.sync_copy(z_hbm, acc_sh.at[pl.ds(s * NPS, NPS)])
    plsc.subcore_barrier()
    xc = xws_hbm.at[c]

    def start_gather(j, b):
        pltpu.async_copy(xc.at[src_v.at[j]], rows_v.at[b], gsem.at[b])

    def wait_gather(b):
        pltpu.make_async_copy(xc.at[src_v.at[0]], rows_v.at[b],
                              gsem.at[b]).wait()

    def start_scatter(j, b):
        pltpu.async_copy(rows_v.at[b], acc_sh.at[dst_v.at[j]], ssem.at[b],
                         add=True)

    def wait_scatter(b):
        pltpu.make_async_copy(rows_v.at[b], acc_sh.at[dst_v.at[0]],
                              ssem.at[b]).wait()

    start_gather(0, 0)
    start_gather(1, 1)

    def body(jj, carry):
        for b in range(4):
            j = 4 * jj + b
            bn = (b + 2) % 4

            @pl.when(j + 2 < NCH)
            def _():
                @pl.when(j >= 2)
                def _():
                    wait_scatter(bn)
                start_gather(j + 2, bn)

            wait_gather(b)
            start_scatter(j, b)
        return carry

    lax.fori_loop(0, NCH // 4, body, 0)
    for b in range(4):
        wait_scatter(b)
    plsc.subcore_barrier()
    pltpu.sync_copy(acc_sh.at[pl.ds(s * NPS, NPS)],
                    out_hbm.at[c, pl.ds(s * NPS, NPS)])


_hop_call = functools.partial(
    pl.kernel,
    _hop_body,
    out_type=jax.ShapeDtypeStruct((NC, NP, DH), jnp.float32),
    mesh=_MESH,
    compiler_params=pltpu.CompilerParams(use_tc_tiling_on_sc=False),
    scratch_types=[
        pltpu.VMEM((NCH, C), jnp.int32),
        pltpu.VMEM((NCH, C), jnp.int32),
        pltpu.VMEM((4, C, DH), jnp.float32),
        pltpu.VMEM_SHARED((NP, DH), jnp.float32),
        pltpu.SemaphoreType.DMA((4,)),
        pltpu.SemaphoreType.DMA((4,)),
    ],
)()


def _full(shape):
    return pl.BlockSpec(shape, lambda i: (0,) * len(shape))


_W = _full((D, D))
_B = _full((1, D))
_ROWBLK = pl.BlockSpec((ROWS, D), lambda i: (i, 0))
_ACCBLK = pl.BlockSpec((NC, ROWS, DH), lambda i: (0, i, 0))
_DEGBLK = pl.BlockSpec((NC, ROWS, 8), lambda i: (0, i, 0))
_DINVBLK = pl.BlockSpec((ROWS, 8), lambda i: (i, 0))


def _stage0_body(x_ref, w1, b1, w2, b2, cw, h_ref, xw_ref):
    h = _mm(_gelu(_mm(x_ref[...], w1[...]) + b1[...]), w2[...]) + b2[...]
    h_ref[...] = h
    xw_ref[...] = _mm(h, cw[...])


_stage0 = pl.pallas_call(
    _stage0_body,
    grid=(N // ROWS,),
    in_specs=[_ROWBLK, _W, _B, _W, _B, _W],
    out_specs=[_ROWBLK, _ROWBLK],
    out_shape=[jax.ShapeDtypeStruct((N, D), jnp.float32)] * 2,
)


def _write_xws(xws_ref, v):
    xws_ref[0] = v[:, :DH]
    xws_ref[1] = v[:, DH:]


def _scale_body(degp, xw, dinv_ref, xws_ref):
    deg = degp[0] + degp[1] + 1.0
    dinv = lax.rsqrt(deg)
    dinv_ref[...] = dinv
    _write_xws(xws_ref, dinv[:, 0:1] * xw[...])


_scale = pl.pallas_call(
    _scale_body,
    grid=(N // ROWS,),
    in_specs=[_DEGBLK, _ROWBLK],
    out_specs=[_DINVBLK, _ACCBLK],
    out_shape=[jax.ShapeDtypeStruct((N, 8), jnp.float32),
               jax.ShapeDtypeStruct((NC, N, DH), jnp.float32)],
)


def _combine_ln_ffn(acc, xw, h_in, dinv, cb, g, b, fw1, fb1, fw2, fb2):
    di = dinv[:, 0:1]
    aggc = jnp.concatenate([acc[0], acc[1]], axis=-1)
    agg = aggc * di + (di * di) * xw + cb
    h1 = agg + h_in
    mu = jnp.mean(h1, axis=-1, keepdims=True)
    cen = h1 - mu
    var = jnp.mean(cen * cen, axis=-1, keepdims=True)
    hn = cen * lax.rsqrt(var + 1e-5) * g + b
    return _mm(_gelu(_mm(hn, fw1) + fb1), fw2) + fb2 + hn


def _hop_post_body(acc, xw, h, dinv, cb, g, b, fw1, fb1, fw2, fb2, cwn,
                   h_out, xw_out, xws_out):
    h2 = _combine_ln_ffn(acc[...], xw[...], h[...], dinv[...], cb[...],
                         g[...], b[...], fw1[...], fb1[...], fw2[...],
                         fb2[...])
    h_out[...] = h2
    xwn = _mm(h2, cwn[...])
    xw_out[...] = xwn
    _write_xws(xws_out, dinv[:, 0:1] * xwn)


_hop_post = pl.pallas_call(
    _hop_post_body,
    grid=(N // ROWS,),
    in_specs=[_ACCBLK, _ROWBLK, _ROWBLK, _DINVBLK,
              _B, _B, _B, _W, _B, _W, _B, _W],
    out_specs=[_ROWBLK, _ROWBLK, _ACCBLK],
    out_shape=[jax.ShapeDtypeStruct((N, D), jnp.float32),
               jax.ShapeDtypeStruct((N, D), jnp.float32),
               jax.ShapeDtypeStruct((NC, N, DH), jnp.float32)],
)


def _final_body(acc, xw, h, dinv, cb, g, b, fw1, fb1, fw2, fb2,
                pw1, pb1, pw2, pb2, out_ref):
    h2 = _combine_ln_ffn(acc[...], xw[...], h[...], dinv[...], cb[...],
                         g[...], b[...], fw1[...], fb1[...], fw2[...],
                         fb2[...])
    out_ref[...] = _mm(_gelu(_mm(h2, pw1[...]) + pb1[...]), pw2[...]) + pb2[...]


_final = pl.pallas_call(
    _final_body,
    grid=(N // ROWS,),
    in_specs=[_ACCBLK, _ROWBLK, _ROWBLK, _DINVBLK,
              _B, _B, _B, _W, _B, _W, _B, _W, _B, _W, _B],
    out_specs=_ROWBLK,
    out_shape=jax.ShapeDtypeStruct((N, D), jnp.float32),
)


def kernel(x, edge_index, pre_W1, pre_b1, pre_W2, pre_b2, conv_W, conv_b,
           ffn_W1, ffn_b1, ffn_W2, ffn_b2, ln_g, ln_b, post_W1, post_b1,
           post_W2, post_b2):
    f32 = jnp.float32
    src3 = edge_index[0].reshape(NS, NCH, C)
    dst3 = edge_index[1].reshape(NS, NCH, C)
    dstd = edge_index[1].reshape(NW, DCH, C)
    zrow = jnp.zeros((NPS, DH), f32)
    z8 = jnp.zeros((NPS, 8), f32)
    ones8 = jnp.ones((C, 8), f32)

    def r1(v):
        return v.reshape(1, D).astype(f32)

    degp = _deg_call(dstd, ones8, z8)
    h, xw = _stage0(x.astype(f32), pre_W1, r1(pre_b1), pre_W2, r1(pre_b2),
                    conv_W[0])
    dinv, xws = _scale(degp, xw)

    for i in range(HOPS_):
        acc = _hop_call(xws, src3, dst3, zrow)
        if i + 1 < HOPS_:
            h, xw, xws = _hop_post(
                acc, xw, h, dinv, r1(conv_b[i]), r1(ln_g[i]), r1(ln_b[i]),
                ffn_W1[i], r1(ffn_b1[i]), ffn_W2[i], r1(ffn_b2[i]),
                conv_W[i + 1])
        else:
            out = _final(
                acc, xw, h, dinv, r1(conv_b[i]), r1(ln_g[i]), r1(ln_b[i]),
                ffn_W1[i], r1(ffn_b1[i]), ffn_W2[i], r1(ffn_b2[i]),
                post_W1, r1(post_b1), post_W2, r1(post_b2))
    return out


# speedup vs baseline: 22.5201x; 1.1012x over previous
import functools

import jax
import jax.numpy as jnp
from jax import lax
from jax.experimental import pallas as pl
from jax.experimental.pallas import tpu as pltpu
from jax.experimental.pallas import tpu_sc as plsc

N = 10000
E = 320000
D = 128
HOPS_ = 5

NC, NS = 2, 16
NW = NC * NS
DH = D // NC
EPS = E // NS
C = 125
NCH = EPS // C
NP = 10240
NPS = NP // NS

ROWS = 1000


def _gelu(v):
    return 0.5 * v * (1.0 + lax.erf(v * 0.7071067811865476))


def _mm(a, b):
    return jax.lax.dot_general(a, b, (((1,), (0,)), ((), ())),
                               preferred_element_type=jnp.float32)


_MESH = plsc.VectorSubcoreMesh(core_axis_name="c", subcore_axis_name="s")


DCH = E // NW // C


def _deg_body(dst_hbm, ones_hbm, z8_hbm, out_hbm, dst_v, ones_v, acc_sh):
    c = lax.axis_index("c")
    s = lax.axis_index("s")
    wid = s * NC + c
    pltpu.sync_copy(dst_hbm.at[wid], dst_v)
    pltpu.sync_copy(ones_hbm, ones_v)
    pltpu.sync_copy(z8_hbm, acc_sh.at[pl.ds(s * NPS, NPS)])
    plsc.subcore_barrier()

    def body(j, carry):
        pltpu.sync_copy(ones_v, acc_sh.at[dst_v.at[j]], add=True)
        return carry

    lax.fori_loop(0, DCH, body, 0)
    plsc.subcore_barrier()
    pltpu.sync_copy(acc_sh.at[pl.ds(s * NPS, NPS)],
                    out_hbm.at[pl.ds(s * NPS, NPS), pl.ds(c * 8, 8)])


_deg_call = functools.partial(
    pl.kernel,
    _deg_body,
    out_type=jax.ShapeDtypeStruct((NP, 16), jnp.float32),
    mesh=_MESH,
    compiler_params=pltpu.CompilerParams(use_tc_tiling_on_sc=False),
    scratch_types=[
        pltpu.VMEM((DCH, C), jnp.int32),
        pltpu.VMEM((C, 8), jnp.float32),
        pltpu.VMEM_SHARED((NP, 8), jnp.float32),
    ],
)()


def _hop_body(xws_hbm, src_hbm, dst_hbm, z_hbm, out_hbm,
              src_v, dst_v, rows_v, acc_sh, gsem, ssem):
    c = lax.axis_index("c")
    s = lax.axis_index("s")
    pltpu.sync_copy(src_hbm.at[s], src_v)
    pltpu.sync_copy(dst_hbm.at[s], dst_v)
    pltpu.sync_copy(z_hbm, acc_sh.at[pl.ds(s * NPS, NPS)])
    plsc.subcore_barrier()
    xc = xws_hbm.at[c]

    def start_gather(j, b):
        pltpu.async_copy(xc.at[src_v.at[j]], rows_v.at[b], gsem.at[b])

    def wait_gather(b):
        pltpu.make_async_copy(xc.at[src_v.at[0]], rows_v.at[b],
                              gsem.at[b]).wait()

    def start_scatter(j, b):
        pltpu.async_copy(rows_v.at[b], acc_sh.at[dst_v.at[j]], ssem.at[b],
                         add=True)

    def wait_scatter(b):
        pltpu.make_async_copy(rows_v.at[b], acc_sh.at[dst_v.at[0]],
                              ssem.at[b]).wait()

    start_gather(0, 0)
    start_gather(1, 1)

    def body(jj, carry):
        for b in range(4):
            j = 4 * jj + b
            bn = (b + 2) % 4

            @pl.when(j + 2 < NCH)
            def _():
                @pl.when(j >= 2)
                def _():
                    wait_scatter(bn)
                start_gather(j + 2, bn)

            wait_gather(b)
            start_scatter(j, b)
        return carry

    lax.fori_loop(0, NCH // 4, body, 0)
    for b in range(4):
        wait_scatter(b)
    plsc.subcore_barrier()
    pltpu.sync_copy(acc_sh.at[pl.ds(s * NPS, NPS)],
                    out_hbm.at[pl.ds(s * NPS, NPS), pl.ds(c * DH, DH)])


_hop_call = functools.partial(
    pl.kernel,
    _hop_body,
    out_type=jax.ShapeDtypeStruct((NP, D), jnp.float32),
    mesh=_MESH,
    compiler_params=pltpu.CompilerParams(use_tc_tiling_on_sc=False),
    scratch_types=[
        pltpu.VMEM((NCH, C), jnp.int32),
        pltpu.VMEM((NCH, C), jnp.int32),
        pltpu.VMEM((4, C, DH), jnp.float32),
        pltpu.VMEM_SHARED((NP, DH), jnp.float32),
        pltpu.SemaphoreType.DMA((4,)),
        pltpu.SemaphoreType.DMA((4,)),
    ],
)()


def _full(shape):
    return pl.BlockSpec(shape, lambda i: (0,) * len(shape))


_W = _full((D, D))
_B = _full((1, D))
_ROWBLK = pl.BlockSpec((ROWS, D), lambda i: (i, 0))
_XSBLK = pl.BlockSpec((NC, ROWS, DH), lambda i: (0, i, 0))
_DEGBLK = pl.BlockSpec((ROWS, 16), lambda i: (i, 0))
_DINVBLK = pl.BlockSpec((ROWS, 8), lambda i: (i, 0))


def _stage0_body(x_ref, w1, b1, w2, b2, h_ref):
    h_ref[...] = _mm(_gelu(_mm(x_ref[...], w1[...]) + b1[...]),
                     w2[...]) + b2[...]


_stage0 = pl.pallas_call(
    _stage0_body,
    grid=(N // ROWS,),
    in_specs=[_ROWBLK, _W, _B, _W, _B],
    out_specs=_ROWBLK,
    out_shape=jax.ShapeDtypeStruct((N, D), jnp.float32),
)


def _write_xws(xws_ref, v):
    xws_ref[0] = v[:, :DH]
    xws_ref[1] = v[:, DH:]


def _scale_body(degp, h, cw, dinv_ref, xws_ref):
    deg = degp[:, 0:8] + degp[:, 8:16] + 1.0
    dinv = lax.rsqrt(deg)
    dinv_ref[...] = dinv
    _write_xws(xws_ref, _mm(dinv[:, 0:1] * h[...], cw[...]))


_scale = pl.pallas_call(
    _scale_body,
    grid=(N // ROWS,),
    in_specs=[_DEGBLK, _ROWBLK, _W],
    out_specs=[_DINVBLK, _XSBLK],
    out_shape=[jax.ShapeDtypeStruct((N, 8), jnp.float32),
               jax.ShapeDtypeStruct((NC, N, DH), jnp.float32)],
)


def _combine_ln_ffn(acc, xws, h_in, dinv, cb, g, b, fw1, fb1, fw2, fb2):
    di = dinv[:, 0:1]
    xwsf = jnp.concatenate([xws[0], xws[1]], axis=-1)
    agg = (acc + xwsf) * di + cb
    h1 = agg + h_in
    mu = jnp.mean(h1, axis=-1, keepdims=True)
    cen = h1 - mu
    var = jnp.mean(cen * cen, axis=-1, keepdims=True)
    hn = cen * lax.rsqrt(var + 1e-5) * g + b
    return _mm(_gelu(_mm(hn, fw1) + fb1), fw2) + fb2 + hn


def _hop_post_body(acc, xws, h, dinv, cb, g, b, fw1, fb1, fw2, fb2, cwn,
                   h_out, xws_out):
    h2 = _combine_ln_ffn(acc[...], xws[...], h[...], dinv[...], cb[...],
                         g[...], b[...], fw1[...], fb1[...], fw2[...],
                         fb2[...])
    h_out[...] = h2
    _write_xws(xws_out, _mm(dinv[:, 0:1] * h2, cwn[...]))


_hop_post = pl.pallas_call(
    _hop_post_body,
    grid=(N // ROWS,),
    in_specs=[_ROWBLK, _XSBLK, _ROWBLK, _DINVBLK,
              _B, _B, _B, _W, _B, _W, _B, _W],
    out_specs=[_ROWBLK, _XSBLK],
    out_shape=[jax.ShapeDtypeStruct((N, D), jnp.float32),
               jax.ShapeDtypeStruct((NC, N, DH), jnp.float32)],
)


def _final_body(acc, xws, h, dinv, cb, g, b, fw1, fb1, fw2, fb2,
                pw1, pb1, pw2, pb2, out_ref):
    h2 = _combine_ln_ffn(acc[...], xws[...], h[...], dinv[...], cb[...],
                         g[...], b[...], fw1[...], fb1[...], fw2[...],
                         fb2[...])
    out_ref[...] = _mm(_gelu(_mm(h2, pw1[...]) + pb1[...]), pw2[...]) + pb2[...]


_final = pl.pallas_call(
    _final_body,
    grid=(N // ROWS,),
    in_specs=[_ROWBLK, _XSBLK, _ROWBLK, _DINVBLK,
              _B, _B, _B, _W, _B, _W, _B, _W, _B, _W, _B],
    out_specs=_ROWBLK,
    out_shape=jax.ShapeDtypeStruct((N, D), jnp.float32),
)


def kernel(x, edge_index, pre_W1, pre_b1, pre_W2, pre_b2, conv_W, conv_b,
           ffn_W1, ffn_b1, ffn_W2, ffn_b2, ln_g, ln_b, post_W1, post_b1,
           post_W2, post_b2):
    f32 = jnp.float32
    src3 = edge_index[0].reshape(NS, NCH, C)
    dst3 = edge_index[1].reshape(NS, NCH, C)
    dstd = edge_index[1].reshape(NW, DCH, C)
    zrow = jnp.zeros((NPS, DH), f32)
    z8 = jnp.zeros((NPS, 8), f32)
    ones8 = jnp.ones((C, 8), f32)

    def r1(v):
        return v.reshape(1, D).astype(f32)

    degp = _deg_call(dstd, ones8, z8)
    h = _stage0(x.astype(f32), pre_W1, r1(pre_b1), pre_W2, r1(pre_b2))
    dinv, xws = _scale(degp, h, conv_W[0])

    for i in range(HOPS_):
        acc = _hop_call(xws, src3, dst3, zrow)
        if i + 1 < HOPS_:
            h, xws = _hop_post(
                acc, xws, h, dinv, r1(conv_b[i]), r1(ln_g[i]), r1(ln_b[i]),
                ffn_W1[i], r1(ffn_b1[i]), ffn_W2[i], r1(ffn_b2[i]),
                conv_W[i + 1])
        else:
            out = _final(
                acc, xws, h, dinv, r1(conv_b[i]), r1(ln_g[i]), r1(ln_b[i]),
                ffn_W1[i], r1(ffn_b1[i]), ffn_W2[i], r1(ffn_b2[i]),
                post_W1, r1(post_b1), post_W2, r1(post_b2))
    return out


# speedup vs baseline: 24.4642x; 1.0863x over previous
import functools

import jax
import jax.numpy as jnp
from jax import lax
from jax.experimental import pallas as pl
from jax.experimental.pallas import tpu as pltpu
from jax.experimental.pallas import tpu_sc as plsc

N = 10000
E = 320000
D = 128
HOPS_ = 5

NC, NS = 2, 16
NW = NC * NS
DH = D // NC
EPS = E // NS
C = 125
NCH = EPS // C
NP = 10240
NPS = NP // NS

ROWS = 1000


def _gelu(v):
    return 0.5 * v * (1.0 + lax.erf(v * 0.7071067811865476))


def _mm(a, b):
    return jax.lax.dot_general(a, b, (((1,), (0,)), ((), ())),
                               preferred_element_type=jnp.float32)


_MESH = plsc.VectorSubcoreMesh(core_axis_name="c", subcore_axis_name="s")


DCH = E // NW // C


def _deg_body(dst_hbm, ones_hbm, z8_hbm, out_hbm, dst_v, ones_v, acc_sh):
    c = lax.axis_index("c")
    s = lax.axis_index("s")
    wid = s * NC + c
    pltpu.sync_copy(dst_hbm.at[wid], dst_v)
    pltpu.sync_copy(ones_hbm, ones_v)
    pltpu.sync_copy(z8_hbm, acc_sh.at[pl.ds(s * NPS, NPS)])
    plsc.subcore_barrier()

    def body(j, carry):
        pltpu.sync_copy(ones_v, acc_sh.at[dst_v.at[j]], add=True)
        return carry

    lax.fori_loop(0, DCH, body, 0)
    plsc.subcore_barrier()
    pltpu.sync_copy(acc_sh.at[pl.ds(s * NPS, NPS)],
                    out_hbm.at[pl.ds(s * NPS, NPS), pl.ds(c * 8, 8)])


_deg_call = functools.partial(
    pl.kernel,
    _deg_body,
    out_type=jax.ShapeDtypeStruct((NP, 16), jnp.float32),
    mesh=_MESH,
    compiler_params=pltpu.CompilerParams(use_tc_tiling_on_sc=False),
    scratch_types=[
        pltpu.VMEM((DCH, C), jnp.int32),
        pltpu.VMEM((C, 8), jnp.float32),
        pltpu.VMEM_SHARED((NP, 8), jnp.float32),
    ],
)()


def _hop_body(xws_hbm, src_hbm, dst_hbm, z_hbm, out_hbm,
              src_v, dst_v, rows_v, acc_sh, gsem, ssem):
    c = lax.axis_index("c")
    s = lax.axis_index("s")
    pltpu.sync_copy(src_hbm.at[s], src_v)
    pltpu.sync_copy(dst_hbm.at[s], dst_v)
    pltpu.sync_copy(z_hbm, acc_sh.at[pl.ds(s * NPS, NPS)])
    plsc.subcore_barrier()
    xc = xws_hbm.at[c]

    def start_gather(j, b):
        pltpu.async_copy(xc.at[src_v.at[j]], rows_v.at[b], gsem.at[b])

    def wait_gather(b):
        pltpu.make_async_copy(xc.at[src_v.at[0]], rows_v.at[b],
                              gsem.at[b]).wait()

    def start_scatter(j, b):
        pltpu.async_copy(rows_v.at[b], acc_sh.at[dst_v.at[j]], ssem.at[b],
                         add=True)

    def wait_scatter(b):
        pltpu.make_async_copy(rows_v.at[b], acc_sh.at[dst_v.at[0]],
                              ssem.at[b]).wait()

    start_gather(0, 0)
    start_gather(1, 1)
    start_gather(2, 2)

    def body(jj, carry):
        for b in range(6):
            j = 6 * jj + b
            bn = (b + 3) % 6

            @pl.when(j + 3 < NCH)
            def _():
                @pl.when(j >= 3)
                def _():
                    wait_scatter(bn)
                start_gather(j + 3, bn)

            wait_gather(b)
            start_scatter(j, b)
        return carry

    lax.fori_loop(0, NCH // 6, body, 0)
    for j in range(NCH - NCH % 6, NCH):
        b = j % 6
        if j + 3 < NCH:
            wait_scatter((b + 3) % 6)
            start_gather(j + 3, (b + 3) % 6)
        wait_gather(b)
        start_scatter(j, b)
    for b in range(6):
        wait_scatter(b)
    plsc.subcore_barrier()
    pltpu.sync_copy(acc_sh.at[pl.ds(s * NPS, NPS)],
                    out_hbm.at[pl.ds(s * NPS, NPS), pl.ds(c * DH, DH)])


_hop_call = functools.partial(
    pl.kernel,
    _hop_body,
    out_type=jax.ShapeDtypeStruct((NP, D), jnp.float32),
    mesh=_MESH,
    compiler_params=pltpu.CompilerParams(use_tc_tiling_on_sc=False),
    scratch_types=[
        pltpu.VMEM((NCH, C), jnp.int32),
        pltpu.VMEM((NCH, C), jnp.int32),
        pltpu.VMEM((6, C, DH), jnp.float32),
        pltpu.VMEM_SHARED((NP, DH), jnp.float32),
        pltpu.SemaphoreType.DMA((6,)),
        pltpu.SemaphoreType.DMA((6,)),
    ],
)()


def _full(shape):
    return pl.BlockSpec(shape, lambda i: (0,) * len(shape))


_W = _full((D, D))
_B = _full((1, D))
_ROWBLK = pl.BlockSpec((ROWS, D), lambda i: (i, 0))
_XSBLK = pl.BlockSpec((NC, ROWS, DH), lambda i: (0, i, 0))
_DEGBLK = pl.BlockSpec((ROWS, 16), lambda i: (i, 0))
_DINVBLK = pl.BlockSpec((ROWS, 8), lambda i: (i, 0))


def _stage0_body(x_ref, w1, b1, w2, b2, h_ref):
    h_ref[...] = _mm(_gelu(_mm(x_ref[...], w1[...]) + b1[...]),
                     w2[...]) + b2[...]


_stage0 = pl.pallas_call(
    _stage0_body,
    grid=(N // ROWS,),
    in_specs=[_ROWBLK, _W, _B, _W, _B],
    out_specs=_ROWBLK,
    out_shape=jax.ShapeDtypeStruct((N, D), jnp.float32),
)


def _write_xws(xws_ref, v):
    xws_ref[0] = v[:, :DH]
    xws_ref[1] = v[:, DH:]


def _scale_body(degp, h, cw, dinv_ref, xws_ref):
    deg = degp[:, 0:8] + degp[:, 8:16] + 1.0
    dinv = lax.rsqrt(deg)
    dinv_ref[...] = dinv
    _write_xws(xws_ref, _mm(dinv[:, 0:1] * h[...], cw[...]))


_scale = pl.pallas_call(
    _scale_body,
    grid=(N // ROWS,),
    in_specs=[_DEGBLK, _ROWBLK, _W],
    out_specs=[_DINVBLK, _XSBLK],
    out_shape=[jax.ShapeDtypeStruct((N, 8), jnp.float32),
               jax.ShapeDtypeStruct((NC, N, DH), jnp.float32)],
)


def _combine_ln_ffn(acc, xws, h_in, dinv, cb, g, b, fw1, fb1, fw2, fb2):
    di = dinv[:, 0:1]
    xwsf = jnp.concatenate([xws[0], xws[1]], axis=-1)
    agg = (acc + xwsf) * di + cb
    h1 = agg + h_in
    mu = jnp.mean(h1, axis=-1, keepdims=True)
    cen = h1 - mu
    var = jnp.mean(cen * cen, axis=-1, keepdims=True)
    hn = cen * lax.rsqrt(var + 1e-5) * g + b
    return _mm(_gelu(_mm(hn, fw1) + fb1), fw2) + fb2 + hn


def _hop_post_body(acc, xws, h, dinv, cb, g, b, fw1, fb1, fw2, fb2, cwn,
                   h_out, xws_out):
    h2 = _combine_ln_ffn(acc[...], xws[...], h[...], dinv[...], cb[...],
                         g[...], b[...], fw1[...], fb1[...], fw2[...],
                         fb2[...])
    h_out[...] = h2
    _write_xws(xws_out, _mm(dinv[:, 0:1] * h2, cwn[...]))


_hop_post = pl.pallas_call(
    _hop_post_body,
    grid=(N // ROWS,),
    in_specs=[_ROWBLK, _XSBLK, _ROWBLK, _DINVBLK,
              _B, _B, _B, _W, _B, _W, _B, _W],
    out_specs=[_ROWBLK, _XSBLK],
    out_shape=[jax.ShapeDtypeStruct((N, D), jnp.float32),
               jax.ShapeDtypeStruct((NC, N, DH), jnp.float32)],
)


def _final_body(acc, xws, h, dinv, cb, g, b, fw1, fb1, fw2, fb2,
                pw1, pb1, pw2, pb2, out_ref):
    h2 = _combine_ln_ffn(acc[...], xws[...], h[...], dinv[...], cb[...],
                         g[...], b[...], fw1[...], fb1[...], fw2[...],
                         fb2[...])
    out_ref[...] = _mm(_gelu(_mm(h2, pw1[...]) + pb1[...]), pw2[...]) + pb2[...]


_final = pl.pallas_call(
    _final_body,
    grid=(N // ROWS,),
    in_specs=[_ROWBLK, _XSBLK, _ROWBLK, _DINVBLK,
              _B, _B, _B, _W, _B, _W, _B, _W, _B, _W, _B],
    out_specs=_ROWBLK,
    out_shape=jax.ShapeDtypeStruct((N, D), jnp.float32),
)


def kernel(x, edge_index, pre_W1, pre_b1, pre_W2, pre_b2, conv_W, conv_b,
           ffn_W1, ffn_b1, ffn_W2, ffn_b2, ln_g, ln_b, post_W1, post_b1,
           post_W2, post_b2):
    f32 = jnp.float32
    src3 = edge_index[0].reshape(NS, NCH, C)
    dst3 = edge_index[1].reshape(NS, NCH, C)
    dstd = edge_index[1].reshape(NW, DCH, C)
    zrow = jnp.zeros((NPS, DH), f32)
    z8 = jnp.zeros((NPS, 8), f32)
    ones8 = jnp.ones((C, 8), f32)

    def r1(v):
        return v.reshape(1, D).astype(f32)

    degp = _deg_call(dstd, ones8, z8)
    h = _stage0(x.astype(f32), pre_W1, r1(pre_b1), pre_W2, r1(pre_b2))
    dinv, xws = _scale(degp, h, conv_W[0])

    for i in range(HOPS_):
        acc = _hop_call(xws, src3, dst3, zrow)
        if i + 1 < HOPS_:
            h, xws = _hop_post(
                acc, xws, h, dinv, r1(conv_b[i]), r1(ln_g[i]), r1(ln_b[i]),
                ffn_W1[i], r1(ffn_b1[i]), ffn_W2[i], r1(ffn_b2[i]),
                conv_W[i + 1])
        else:
            out = _final(
                acc, xws, h, dinv, r1(conv_b[i]), r1(ln_g[i]), r1(ln_b[i]),
                ffn_W1[i], r1(ffn_b1[i]), ffn_W2[i], r1(ffn_b2[i]),
                post_W1, r1(post_b1), post_W2, r1(post_b2))
    return out
